# baseline (speedup 1.0000x reference)
import jax
import jax.numpy as jnp
from jax.experimental import pallas as pl
from jax.experimental.pallas import tpu as pltpu

_SUB = 256
_K = 8


def _fused_lowrank_kernel(x_hbm, win_ref, wout_ref, out_hbm,
                          xbuf, obuf, lsem, ssem):
    n_tokens = x_hbm.shape[0]
    n_steps = n_tokens // _SUB
    n_groups = n_steps // _K
    win = win_ref[...]
    wout = wout_ref[...]

    def load(step, slot):
        return pltpu.make_async_copy(
            x_hbm.at[pl.ds(step * _SUB, _SUB), :], xbuf.at[slot],
            lsem.at[slot])

    def store(step, slot):
        return pltpu.make_async_copy(
            obuf.at[slot], out_hbm.at[pl.ds(step * _SUB, _SUB), :],
            ssem.at[slot])

    def compute(slot):
        h = jnp.dot(xbuf[slot], win, preferred_element_type=jnp.float32)
        obuf[slot] = jnp.dot(h, wout, preferred_element_type=jnp.float32)

    for k in range(_K):
        load(k, k).start()

    for k in range(_K):
        load(k, k).wait()
        compute(k)
        store(k, k).start()
        load(k + _K, k).start()

    def group_body(g, carry):
        base = g * _K
        for k in range(_K):
            step = base + k
            load(step, k).wait()
            store(step - _K, k).wait()
            compute(k)
            store(step, k).start()
            load(step + _K, k).start()
        return carry

    jax.lax.fori_loop(1, n_groups - 1, group_body, 0)

    base = (n_groups - 1) * _K
    for k in range(_K):
        load(base + k, k).wait()
        store(base + k - _K, k).wait()
        compute(k)
        store(base + k, k).start()

    for k in range(_K):
        store(base + k, k).wait()


@jax.jit
def kernel(x, shared_in, shared_out):
    n_tokens, in_features = x.shape
    rank, out_features = shared_out.shape

    return pl.pallas_call(
        _fused_lowrank_kernel,
        in_specs=[
            pl.BlockSpec(memory_space=pltpu.MemorySpace.HBM),
            pl.BlockSpec(memory_space=pltpu.MemorySpace.VMEM),
            pl.BlockSpec(memory_space=pltpu.MemorySpace.VMEM),
        ],
        out_specs=pl.BlockSpec(memory_space=pltpu.MemorySpace.HBM),
        out_shape=jax.ShapeDtypeStruct((n_tokens, out_features), jnp.float32),
        scratch_shapes=[
            pltpu.VMEM((_K, _SUB, in_features), jnp.float32),
            pltpu.VMEM((_K, _SUB, out_features), jnp.float32),
            pltpu.SemaphoreType.DMA((_K,)),
            pltpu.SemaphoreType.DMA((_K,)),
        ],
    )(x, shared_in, shared_out)


# speedup vs baseline: 1.0410x; 1.0410x over previous
import jax
import jax.numpy as jnp
from jax.experimental import pallas as pl
from jax.experimental.pallas import tpu as pltpu

_SUB = 512
_KL = 8
_KS = 4


def _fused_lowrank_kernel(x_hbm, win_ref, wout_ref, out_hbm,
                          xbuf, obuf, lsem, ssem):
    n_tokens = x_hbm.shape[0]
    n_steps = n_tokens // _SUB
    n_groups = n_steps // _KL
    win = win_ref[...]
    wout = wout_ref[...]

    def load(step, slot):
        return pltpu.make_async_copy(
            x_hbm.at[pl.ds(step * _SUB, _SUB), :], xbuf.at[slot],
            lsem.at[slot])

    def store(step, slot):
        return pltpu.make_async_copy(
            obuf.at[slot], out_hbm.at[pl.ds(step * _SUB, _SUB), :],
            ssem.at[slot])

    def compute(lslot, oslot):
        h = jnp.dot(xbuf[lslot], win, preferred_element_type=jnp.float32)
        obuf[oslot] = jnp.dot(h, wout, preferred_element_type=jnp.float32)

    for k in range(_KL):
        load(k, k).start()

    for k in range(_KL):
        load(k, k).wait()
        if k >= _KS:
            store(k - _KS, k % _KS).wait()
        compute(k, k % _KS)
        store(k, k % _KS).start()
        load(k + _KL, k).start()

    def group_body(g, carry):
        base = g * _KL
        for k in range(_KL):
            step = base + k
            load(step, k).wait()
            store(step - _KS, step % _KS).wait()
            compute(k, step % _KS)
            store(step, step % _KS).start()
            load(step + _KL, k).start()
        return carry

    jax.lax.fori_loop(1, n_groups - 1, group_body, 0)

    base = (n_groups - 1) * _KL
    for k in range(_KL):
        step = base + k
        load(step, k).wait()
        store(step - _KS, step % _KS).wait()
        compute(k, step % _KS)
        store(step, step % _KS).start()

    for step in range(n_steps - _KS, n_steps):
        store(step, step % _KS).wait()


@jax.jit
def kernel(x, shared_in, shared_out):
    n_tokens, in_features = x.shape
    rank, out_features = shared_out.shape

    return pl.pallas_call(
        _fused_lowrank_kernel,
        in_specs=[
            pl.BlockSpec(memory_space=pltpu.MemorySpace.HBM),
            pl.BlockSpec(memory_space=pltpu.MemorySpace.VMEM),
            pl.BlockSpec(memory_space=pltpu.MemorySpace.VMEM),
        ],
        out_specs=pl.BlockSpec(memory_space=pltpu.MemorySpace.HBM),
        out_shape=jax.ShapeDtypeStruct((n_tokens, out_features), jnp.float32),
        scratch_shapes=[
            pltpu.VMEM((_KL, _SUB, in_features), jnp.float32),
            pltpu.VMEM((_KS, _SUB, out_features), jnp.float32),
            pltpu.SemaphoreType.DMA((_KL,)),
            pltpu.SemaphoreType.DMA((_KS,)),
        ],
    )(x, shared_in, shared_out)


# speedup vs baseline: 1.0520x; 1.0106x over previous
import jax
import jax.numpy as jnp
from jax.experimental import pallas as pl
from jax.experimental.pallas import tpu as pltpu

_SUB = 512
_KL = 4
_KS = 4


def _fused_lowrank_kernel(x_hbm, win_hbm, wout_hbm, out_hbm,
                          xbuf, obuf, win_vmem, wout_vmem, lsem, ssem, wsem):
    n_tokens = x_hbm.shape[0]
    n_steps = n_tokens // _SUB
    n_groups = n_steps // _KL

    win_copy = pltpu.make_async_copy(win_hbm, win_vmem, wsem.at[0])
    wout_copy = pltpu.make_async_copy(wout_hbm, wout_vmem, wsem.at[1])
    win_copy.start()
    wout_copy.start()

    def load(step, slot):
        return pltpu.make_async_copy(
            x_hbm.at[pl.ds(step * _SUB, _SUB), :], xbuf.at[slot],
            lsem.at[slot])

    def store(step, slot):
        return pltpu.make_async_copy(
            obuf.at[slot], out_hbm.at[pl.ds(step * _SUB, _SUB), :],
            ssem.at[slot])

    def compute(lslot, oslot):
        h = jnp.dot(xbuf[lslot], win_vmem[...],
                    preferred_element_type=jnp.float32)
        obuf[oslot] = jnp.dot(h, wout_vmem[...],
                              preferred_element_type=jnp.float32)

    for k in range(_KL):
        load(k, k).start()
    win_copy.wait()
    wout_copy.wait()

    for k in range(_KL):
        load(k, k).wait()
        if k >= _KS:
            store(k - _KS, k % _KS).wait()
        compute(k, k % _KS)
        store(k, k % _KS).start()
        load(k + _KL, k).start()

    def group_body(g, carry):
        base = g * _KL
        for k in range(_KL):
            step = base + k
            load(step, k).wait()
            store(step - _KS, step % _KS).wait()
            compute(k, step % _KS)
            store(step, step % _KS).start()
            load(step + _KL, k).start()
        return carry

    jax.lax.fori_loop(1, n_groups - 1, group_body, 0)

    base = (n_groups - 1) * _KL
    for k in range(_KL):
        step = base + k
        load(step, k).wait()
        store(step - _KS, step % _KS).wait()
        compute(k, step % _KS)
        store(step, step % _KS).start()

    for step in range(n_steps - _KS, n_steps):
        store(step, step % _KS).wait()


@jax.jit
def kernel(x, shared_in, shared_out):
    n_tokens, in_features = x.shape
    rank, out_features = shared_out.shape

    return pl.pallas_call(
        _fused_lowrank_kernel,
        in_specs=[
            pl.BlockSpec(memory_space=pltpu.MemorySpace.HBM),
            pl.BlockSpec(memory_space=pltpu.MemorySpace.HBM),
            pl.BlockSpec(memory_space=pltpu.MemorySpace.HBM),
        ],
        out_specs=pl.BlockSpec(memory_space=pltpu.MemorySpace.HBM),
        out_shape=jax.ShapeDtypeStruct((n_tokens, out_features), jnp.float32),
        scratch_shapes=[
            pltpu.VMEM((_KL, _SUB, in_features), jnp.float32),
            pltpu.VMEM((_KS, _SUB, out_features), jnp.float32),
            pltpu.VMEM((in_features, rank), jnp.float32),
            pltpu.VMEM((rank, out_features), jnp.float32),
            pltpu.SemaphoreType.DMA((_KL,)),
            pltpu.SemaphoreType.DMA((_KS,)),
            pltpu.SemaphoreType.DMA((2,)),
        ],
    )(x, shared_in, shared_out)
